# baseline (speedup 1.0000x reference)
import functools

import jax
import jax.numpy as jnp
from jax import lax
from jax.experimental import pallas as pl
from jax.experimental.pallas import tpu as pltpu
from jax.experimental.pallas import tpu_sc as plsc

_L = 16


def _make_pe_add(B, T, D):
    info = plsc.get_sparse_core_info()
    NC, NS = info.num_cores, info.num_subcores
    NW = NC * NS
    TW = T // NW
    CH = 4
    NCH = TW // CH
    NBLK = D // _L

    mesh = plsc.VectorSubcoreMesh(core_axis_name="c", subcore_axis_name="s")

    @functools.partial(
        pl.kernel,
        out_type=jax.ShapeDtypeStruct((B, T, D), jnp.float32),
        mesh=mesh,
        scratch_types=(
            [pltpu.VMEM((CH, D), jnp.float32)] * (4 * B + 2)
            + [pltpu.SemaphoreType.DMA] * 6
        ),
    )
    def pe_add(x_hbm, pe_hbm, out_hbm, *bufs):
        ins = (bufs[0:B], bufs[B:2 * B])
        outs = (bufs[2 * B:3 * B], bufs[3 * B:4 * B])
        pes = bufs[4 * B:4 * B + 2]
        sin = bufs[4 * B + 2:4 * B + 4]
        sout = bufs[4 * B + 4:4 * B + 6]
        spe = bufs[4 * B + 6:4 * B + 8]

        wid = lax.axis_index("s") * NC + lax.axis_index("c")
        t_base = wid * TW

        def x_in(c, b, s):
            return pltpu.make_async_copy(
                x_hbm.at[b, pl.ds(t_base + c * CH, CH), :], ins[s][b], sin[s])

        def pe_copy(c, s):
            return pltpu.make_async_copy(
                pe_hbm.at[pl.ds(t_base + c * CH, CH), :], pes[s], spe[s])

        def x_out(c, b, s):
            return pltpu.make_async_copy(
                outs[s][b], out_hbm.at[b, pl.ds(t_base + c * CH, CH), :],
                sout[s])

        for s in range(2):
            pe_copy(s, s).start()
            for b in range(B):
                x_in(s, b, s).start()

        def pair_body(g, carry):
            for j in range(2):
                c = 2 * g + j
                s = j
                for b in range(B):
                    x_in(c, b, s).wait()
                pe_copy(c, s).wait()
                @pl.when(g > 0)
                def _():
                    for b in range(B):
                        x_out(c - 2, b, s).wait()

                ib, ob, pb = ins[s], outs[s], pes[s]

                @plsc.parallel_loop(0, CH * NBLK, unroll=8)
                def _(i):
                    r = i // NBLK
                    sl = pl.ds((i % NBLK) * _L, _L)
                    pv = pb[r, sl]
                    for b in range(B):
                        ob[b][r, sl] = ib[b][r, sl] + pv
                for b in range(B):
                    x_out(c, b, s).start()
                @pl.when(c + 2 < NCH)
                def _():
                    pe_copy(c + 2, s).start()
                    for b in range(B):
                        x_in(c + 2, b, s).start()
            return carry

        lax.fori_loop(0, NCH // 2, pair_body, 0)

        for s in range(2):
            for b in range(B):
                x_out(NCH - 2 + s, b, s).wait()

    return pe_add


def kernel(x, posem):
    B, T, D = x.shape
    pe_add = _make_pe_add(B, T, D)
    return pe_add(x, posem)


# speedup vs baseline: 1.0033x; 1.0033x over previous
import functools

import jax
import jax.numpy as jnp
from jax import lax
from jax.experimental import pallas as pl
from jax.experimental.pallas import tpu as pltpu
from jax.experimental.pallas import tpu_sc as plsc

_L = 16


def _make_pe_add(B, T, D):
    info = plsc.get_sparse_core_info()
    NC, NS = info.num_cores, info.num_subcores
    NW = NC * NS
    TW = T // NW
    CH = 4
    NCH = TW // CH
    NBLK = D // _L

    mesh = plsc.VectorSubcoreMesh(core_axis_name="c", subcore_axis_name="s")

    @functools.partial(
        pl.kernel,
        out_type=jax.ShapeDtypeStruct((B, T, D), jnp.float32),
        mesh=mesh,
        scratch_types=(
            [pltpu.VMEM((CH, D), jnp.float32)] * (4 * B + 2)
            + [pltpu.SemaphoreType.DMA] * 6
        ),
    )
    def pe_add(x_hbm, pe_hbm, out_hbm, *bufs):
        ins = (bufs[0:B], bufs[B:2 * B])
        outs = (bufs[2 * B:3 * B], bufs[3 * B:4 * B])
        pes = bufs[4 * B:4 * B + 2]
        sin = bufs[4 * B + 2:4 * B + 4]
        sout = bufs[4 * B + 4:4 * B + 6]
        spe = bufs[4 * B + 6:4 * B + 8]

        wid = lax.axis_index("s") * NC + lax.axis_index("c")
        t_base = wid * TW

        def x_in(c, b, s):
            return pltpu.make_async_copy(
                x_hbm.at[b, pl.ds(t_base + c * CH, CH), :], ins[s][b], sin[s])

        def pe_copy(c, s):
            return pltpu.make_async_copy(
                pe_hbm.at[pl.ds(t_base + c * CH, CH), :], pes[s], spe[s])

        def x_out(c, b, s):
            return pltpu.make_async_copy(
                outs[s][b], out_hbm.at[b, pl.ds(t_base + c * CH, CH), :],
                sout[s])

        for s in range(2):
            pe_copy(s, s).start()
            for b in range(B):
                x_in(s, b, s).start()

        def pair_body(g, carry):
            for j in range(2):
                c = 2 * g + j
                s = j
                for b in range(B):
                    x_in(c, b, s).wait()
                pe_copy(c, s).wait()
                @pl.when(g > 0)
                def _():
                    for b in range(B):
                        x_out(c - 2, b, s).wait()

                ib, ob, pb = ins[s], outs[s], pes[s]

                @plsc.parallel_loop(0, CH * NBLK, unroll=4)
                def _(i):
                    r = i // NBLK
                    sl = pl.ds((i % NBLK) * _L, _L)
                    pv = pb[r, sl]
                    for b in range(B):
                        ob[b][r, sl] = ib[b][r, sl] + pv
                for b in range(B):
                    x_out(c, b, s).start()
                @pl.when(c + 2 < NCH)
                def _():
                    pe_copy(c + 2, s).start()
                    for b in range(B):
                        x_in(c + 2, b, s).start()
            return carry

        lax.fori_loop(0, NCH // 2, pair_body, 0)

        for s in range(2):
            for b in range(B):
                x_out(NCH - 2 + s, b, s).wait()

    return pe_add


def kernel(x, posem):
    B, T, D = x.shape
    pe_add = _make_pe_add(B, T, D)
    return pe_add(x, posem)


# speedup vs baseline: 1.0073x; 1.0040x over previous
import functools

import jax
import jax.numpy as jnp
from jax import lax
from jax.experimental import pallas as pl
from jax.experimental.pallas import tpu as pltpu
from jax.experimental.pallas import tpu_sc as plsc

_L = 16


def _make_pe_add(B, T, D):
    info = plsc.get_sparse_core_info()
    NC, NS = info.num_cores, info.num_subcores
    NW = NC * NS
    TW = T // NW
    CH = 8
    NCH = TW // CH
    NBLK = D // _L
    R = 4

    mesh = plsc.VectorSubcoreMesh(core_axis_name="c", subcore_axis_name="s")

    @functools.partial(
        pl.kernel,
        out_type=jax.ShapeDtypeStruct((B, T, D), jnp.float32),
        mesh=mesh,
        scratch_types=(
            [pltpu.VMEM((CH, D), jnp.float32)] * (2 * R + 2)
            + [pltpu.SemaphoreType.DMA] * (2 * R + 2)
        ),
    )
    def pe_add(x_hbm, pe_hbm, out_hbm, *bufs):
        ins = bufs[0:R]
        outs = bufs[R:2 * R]
        pes = bufs[2 * R:2 * R + 2]
        isems = bufs[2 * R + 2:3 * R + 2]
        osems = bufs[3 * R + 2:4 * R + 2]
        psems = bufs[4 * R + 2:4 * R + 4]

        wid = lax.axis_index("s") * NC + lax.axis_index("c")
        t_base = wid * TW

        def x_copy(c, b, s):
            return pltpu.make_async_copy(
                x_hbm.at[b, pl.ds(t_base + c * CH, CH), :], ins[s], isems[s])

        def pe_copy(c, half):
            return pltpu.make_async_copy(
                pe_hbm.at[pl.ds(t_base + c * CH, CH), :], pes[half],
                psems[half])

        def out_copy(c, b, s):
            return pltpu.make_async_copy(
                outs[s], out_hbm.at[b, pl.ds(t_base + c * CH, CH), :],
                osems[s])

        pe_copy(0, 0).start()
        pe_copy(1, 1).start()
        for j in range(R):
            x_copy(0, j, j).start()

        def pair_body(g, carry):
            for j in range(2 * B):
                c = 2 * g + j // B
                b = j % B
                s = j % R
                half = j // B
                ib, ob, pb = ins[s], outs[s], pes[half]
                x_copy(c, b, s).wait()
                if b == 0:
                    pe_copy(c, half).wait()
                if j >= R:
                    out_copy(c, b, s).wait()
                else:
                    @pl.when(g > 0)
                    def _():
                        out_copy(c, b, s).wait()

                @plsc.parallel_loop(0, CH * NBLK, unroll=4)
                def _(i):
                    r = i // NBLK
                    sl = pl.ds((i % NBLK) * _L, _L)
                    ob[r, sl] = ib[r, sl] + pb[r, sl]
                out_copy(c, b, s).start()
                if j + R < 2 * B:
                    x_copy(c + 1, b, s).start()
                else:
                    @pl.when(g + 1 < NCH // 2)
                    def _():
                        x_copy(c + 1, b, s).start()
                if b == B - 1:
                    @pl.when(c + 2 < NCH)
                    def _():
                        pe_copy(c + 2, half).start()
            return carry

        lax.fori_loop(0, NCH // 2, pair_body, 0)

        for j in range(R):
            out_copy(NCH - 1, j, j).wait()

    return pe_add


def kernel(x, posem):
    B, T, D = x.shape
    pe_add = _make_pe_add(B, T, D)
    return pe_add(x, posem)
